# baseline (speedup 1.0000x reference)
import functools

import jax
import jax.numpy as jnp
from jax import lax
from jax.experimental import pallas as pl
from jax.experimental.pallas import tpu as pltpu
from jax.experimental.pallas import tpu_sc as plsc

_N = 10000
_E = 320000
_DH = 64
_NC, _NS = 2, 16
_NW = _NC * _NS
_K = 128
_ET = _E // _NW
_CF = _ET // _K
_TAIL = _ET - _CF * _K
_NB = 4
_CBF = 19
_TPT = _N // _NS
_DHH = _DH // 2
_NPAD = 10112
_RPT = _NPAD // _NS
_DW = 16

_sc_mesh = plsc.VectorSubcoreMesh(
    core_axis_name="c", subcore_axis_name="s",
    num_cores=_NC, num_subcores=_NS)


@functools.partial(
    pl.kernel,
    out_type=jax.ShapeDtypeStruct((_NC, _NPAD, _DHH), jnp.float32),
    mesh=_sc_mesh,
    scratch_types=[
        pltpu.VMEM((_ET,), jnp.int32),
        pltpu.VMEM((_ET,), jnp.int32),
        [pltpu.VMEM((_K, _DHH), jnp.float32)] * _NB,
        pltpu.VMEM_SHARED((_NPAD, _DHH), jnp.float32),
        pltpu.VMEM_SHARED((_N, _DHH), jnp.float32),
        [pltpu.SemaphoreType.DMA] * _NB,
        [pltpu.SemaphoreType.DMA] * _NB,
    ],
    compiler_params=pltpu.CompilerParams(use_tc_tiling_on_sc=False),
)
def _sc_prop(ei_hbm, tab_hbm, zero_hbm, out_hbm,
             sidx, didx, rows, acc, stab, gs, ss):
    c = lax.axis_index("c")
    s = lax.axis_index("s")
    w = c * _NS + s

    pltpu.sync_copy(ei_hbm.at[0, pl.ds(w * _ET, _ET)], sidx)
    pltpu.sync_copy(ei_hbm.at[1, pl.ds(w * _ET, _ET)], didx)

    def gather(i, b):
        pltpu.async_copy(stab.at[sidx.at[pl.ds(i * _K, _K)]],
                         rows[b], gs[b])

    def scat(i, b):
        pltpu.async_copy(rows[b], acc.at[didx.at[pl.ds(i * _K, _K)]],
                         ss[b], add=True)

    def wait_gather(b):
        pltpu.make_async_copy(stab.at[sidx.at[pl.ds(0, _K)]],
                              rows[b], gs[b]).wait()

    def wait_scat(b):
        pltpu.make_async_copy(rows[b], acc.at[didx.at[pl.ds(0, _K)]],
                              ss[b]).wait()

    def tail_gather():
        pltpu.async_copy(stab.at[sidx.at[pl.ds(_CF * _K, _TAIL)]],
                         rows[2].at[pl.ds(0, _TAIL)], gs[2])

    def wait_tail_gather():
        pltpu.make_async_copy(stab.at[sidx.at[pl.ds(_CF * _K, _TAIL)]],
                              rows[2].at[pl.ds(0, _TAIL)], gs[2]).wait()

    def tail_scat():
        pltpu.async_copy(rows[2].at[pl.ds(0, _TAIL)],
                         acc.at[didx.at[pl.ds(_CF * _K, _TAIL)]],
                         ss[2], add=True)

    def wait_tail_scat():
        pltpu.make_async_copy(rows[2].at[pl.ds(0, _TAIL)],
                              acc.at[didx.at[pl.ds(_CF * _K, _TAIL)]],
                              ss[2]).wait()

    def step(j, carry):
        for t in range(_NB):
            i = _NB * j + t
            b = t
            bp = (t + _NB - 1) % _NB
            wait_gather(b)
            scat(i, b)
            if t == 0:
                pl.when(j > 0)(lambda bp=bp: wait_scat(bp))
                gather(i + _NB - 1, bp)
            elif t < _NB - 1:
                wait_scat(bp)
                gather(i + _NB - 1, bp)
            else:
                def pre(i=i, bp=bp):
                    wait_scat(bp)
                    gather(i + _NB - 1, bp)
                pl.when(j < _CBF - 1)(pre)
        return carry

    pltpu.sync_copy(zero_hbm, acc.at[pl.ds(s * _RPT, _RPT)])
    pltpu.sync_copy(tab_hbm.at[pl.ds(s * _TPT, _TPT)],
                    stab.at[pl.ds(s * _TPT, _TPT)])
    plsc.subcore_barrier()
    for i in range(_NB - 1):
        gather(i, i)
    lax.fori_loop(0, _CBF, step, 0)
    wait_gather(0)
    scat(_CF - 2, 0)
    wait_gather(1)
    scat(_CF - 1, 1)
    wait_scat(2)
    tail_gather()
    wait_tail_gather()
    tail_scat()
    wait_scat(3)
    wait_scat(0)
    wait_scat(1)
    wait_tail_scat()
    plsc.subcore_barrier()
    for k in range(4):
        off = s * _RPT + k * _K
        pltpu.sync_copy(acc.at[pl.ds(off, _K)], rows[k])
        pltpu.sync_copy(rows[k], out_hbm.at[c, pl.ds(off, _K)])
    off = s * _RPT + 4 * _K
    rem = _RPT - 4 * _K
    pltpu.sync_copy(acc.at[pl.ds(off, rem)], rows[0].at[pl.ds(0, rem)])
    pltpu.sync_copy(rows[0].at[pl.ds(0, rem)],
                    out_hbm.at[c, pl.ds(off, rem)])


@functools.partial(
    pl.kernel,
    out_type=jax.ShapeDtypeStruct((_NC, _NPAD, _DW), jnp.float32),
    mesh=_sc_mesh,
    scratch_types=[
        pltpu.VMEM((_ET,), jnp.int32),
        pltpu.VMEM((_K, _DW), jnp.float32),
        pltpu.VMEM_SHARED((_NPAD, _DW), jnp.float32),
        pltpu.SemaphoreType.DMA,
        pltpu.SemaphoreType.DMA,
    ],
    compiler_params=pltpu.CompilerParams(use_tc_tiling_on_sc=False),
)
def _sc_deg(ei_hbm, zero_hbm, ones_hbm, out_hbm, didx, ones_v, acc, s0, s1):
    c = lax.axis_index("c")
    s = lax.axis_index("s")
    w = c * _NS + s

    pltpu.sync_copy(ei_hbm.at[1, pl.ds(w * _ET, _ET)], didx)
    pltpu.sync_copy(zero_hbm, acc.at[pl.ds(s * _RPT, _RPT)])
    pltpu.sync_copy(ones_hbm, ones_v)
    plsc.subcore_barrier()

    def scat(i, sem):
        pltpu.async_copy(ones_v, acc.at[didx.at[pl.ds(i * _K, _K)]],
                         sem, add=True)

    def wait_scat(sem):
        pltpu.make_async_copy(ones_v, acc.at[didx.at[pl.ds(0, _K)]],
                              sem).wait()

    scat(0, s0)
    scat(1, s1)

    def step(j, carry):
        wait_scat(s0)
        scat(2 * j, s0)
        wait_scat(s1)
        scat(2 * j + 1, s1)
        return carry

    lax.fori_loop(1, _CF // 2, step, 0)
    wait_scat(s0)
    pltpu.async_copy(ones_v.at[pl.ds(0, _TAIL)],
                     acc.at[didx.at[pl.ds(_CF * _K, _TAIL)]], s0, add=True)
    wait_scat(s1)
    pltpu.make_async_copy(ones_v.at[pl.ds(0, _TAIL)],
                          acc.at[didx.at[pl.ds(_CF * _K, _TAIL)]], s0).wait()
    plsc.subcore_barrier()
    pltpu.sync_copy(acc.at[pl.ds(s * _RPT, _RPT)],
                    out_hbm.at[c, pl.ds(s * _RPT, _RPT)])


def _tc(body, out_shape, *args):
    return pl.pallas_call(body, out_shape=out_shape)(*args)


def _mm1_body(x, w1, out_u0, out_u1):
    hp = jnp.dot(x[...], w1[...], preferred_element_type=jnp.float32)
    out_u0[...] = hp[:, :_DHH]
    out_u1[...] = hp[:, _DHH:]


def _scale_body(degp, hp0, hp1, out_u0, out_u1, out_dinv):
    deg = degp[0][:_N, 0:1] + degp[1][:_N, 0:1] + 1.0
    dinv = lax.rsqrt(deg)
    out_u0[...] = dinv * hp0[...]
    out_u1[...] = dinv * hp1[...]
    out_dinv[...] = dinv


def _half_tail(h, s_pair, u, dinv, b, g, be):
    col = slice(h * _DHH, (h + 1) * _DHH)
    z = dinv[...] * (s_pair[0][:_N] + s_pair[1][:_N] + u[...]) \
        + b[...][None, col]
    a = jnp.where(z >= 0, z, 0.01 * z)
    m = jnp.mean(a, axis=0, keepdims=True)
    v = jnp.mean((a - m) ** 2, axis=0, keepdims=True)
    return g[...][None, col] * (a - m) * lax.rsqrt(v + 1e-5) \
        + be[...][None, col]


def _tail0_body(s_pair, u, dinv, b, g, be, out_h):
    out_h[...] = _half_tail(0, s_pair, u, dinv, b, g, be)


def _tail1_mm_body(s_pair, u, dinv, b, g, be, h0, wn, out_u0, out_u1):
    h1 = _half_tail(1, s_pair, u, dinv, b, g, be)
    h = jnp.concatenate([h0[...], h1], axis=1)
    un = dinv[...] * jnp.dot(h, wn[...], preferred_element_type=jnp.float32)
    out_u0[...] = un[:, :_DHH]
    out_u1[...] = un[:, _DHH:]


def _tailq0_body(s_pair, u, dinv, b, g, be, out_q):
    out_q[...] = dinv[...] * _half_tail(0, s_pair, u, dinv, b, g, be)


def _tailq1_body(s_pair, u, dinv, b, g, be, out_q):
    out_q[...] = dinv[...] * _half_tail(1, s_pair, u, dinv, b, g, be)


def _r_body(t_pair, q, dinv, out_r):
    out_r[...] = dinv[...] * (t_pair[0][:_N] + t_pair[1][:_N] + q[...])


def _heads_mm_body(r0, r1, wmu, bmu, wls, bls, out_mu, out_ls):
    r = jnp.concatenate([r0[...], r1[...]], axis=1)
    out_mu[...] = jnp.dot(r, wmu[...], preferred_element_type=jnp.float32) \
        + bmu[...][None, :]
    out_ls[...] = jnp.dot(r, wls[...], preferred_element_type=jnp.float32) \
        + bls[...][None, :]


_f32 = jnp.float32


def kernel(x, edge_index, W1, b1, g1, be1, W2, b2, g2, be2,
           W3, b3, g3, be3, Wmu, bmu, Wls, bls):
    zeros32 = jnp.zeros((_RPT, _DHH), _f32)
    zeros16 = jnp.zeros((_RPT, _DW), _f32)
    ones16 = jnp.ones((_K, _DW), _f32)

    nh = jax.ShapeDtypeStruct((_N, _DHH), _f32)
    n1 = jax.ShapeDtypeStruct((_N, 1), _f32)

    degp = _sc_deg(edge_index, zeros16, ones16)
    hp0, hp1 = _tc(_mm1_body, (nh, nh), x, W1)
    u10, u11, dinv = _tc(_scale_body, (nh, nh, n1), degp, hp0, hp1)

    sA = _sc_prop(edge_index, u10, zeros32)
    sB = _sc_prop(edge_index, u11, zeros32)
    h20 = _tc(_tail0_body, nh, sA, u10, dinv, b1, g1, be1)
    u20, u21 = _tc(_tail1_mm_body, (nh, nh), sB, u11, dinv, b1, g1, be1,
                   h20, W2)

    sA = _sc_prop(edge_index, u20, zeros32)
    sB = _sc_prop(edge_index, u21, zeros32)
    h30 = _tc(_tail0_body, nh, sA, u20, dinv, b2, g2, be2)
    u30, u31 = _tc(_tail1_mm_body, (nh, nh), sB, u21, dinv, b2, g2, be2,
                   h30, W3)

    sA = _sc_prop(edge_index, u30, zeros32)
    sB = _sc_prop(edge_index, u31, zeros32)
    q0 = _tc(_tailq0_body, nh, sA, u30, dinv, b3, g3, be3)
    q1 = _tc(_tailq1_body, nh, sB, u31, dinv, b3, g3, be3)

    tA = _sc_prop(edge_index, q0, zeros32)
    tB = _sc_prop(edge_index, q1, zeros32)
    r0 = _tc(_r_body, nh, tA, q0, dinv)
    r1 = _tc(_r_body, nh, tB, q1, dinv)
    no = jax.ShapeDtypeStruct((_N, 128), _f32)
    mu, ls = _tc(_heads_mm_body, (no, no), r0, r1, Wmu, bmu, Wls, bls)
    return (mu, ls)


# speedup vs baseline: 1.1089x; 1.1089x over previous
import functools

import jax
import jax.numpy as jnp
from jax import lax
from jax.experimental import pallas as pl
from jax.experimental.pallas import tpu as pltpu
from jax.experimental.pallas import tpu_sc as plsc

_N = 10000
_E = 320000
_DH = 64
_NC, _NS = 2, 16
_NW = _NC * _NS
_K = 128
_ET = _E // _NW
_CF = _ET // _K
_TAIL = _ET - _CF * _K
_NB = 4
_CBF = 19
_TPT = _N // _NS
_DHH = _DH // 2
_NPAD = 10112
_RPT = _NPAD // _NS
_DW = 16

_sc_mesh = plsc.VectorSubcoreMesh(
    core_axis_name="c", subcore_axis_name="s",
    num_cores=_NC, num_subcores=_NS)


@functools.partial(
    pl.kernel,
    out_type=jax.ShapeDtypeStruct((_NC, _NPAD, _DH), jnp.float32),
    mesh=_sc_mesh,
    scratch_types=[
        pltpu.VMEM((_ET,), jnp.int32),
        pltpu.VMEM((_ET,), jnp.int32),
        [pltpu.VMEM((_K, _DHH), jnp.float32)] * _NB,
        pltpu.VMEM_SHARED((_NPAD, _DHH), jnp.float32),
        pltpu.VMEM_SHARED((_N, _DHH), jnp.float32),
        [pltpu.SemaphoreType.DMA] * _NB,
        [pltpu.SemaphoreType.DMA] * _NB,
    ],
    compiler_params=pltpu.CompilerParams(use_tc_tiling_on_sc=False),
)
def _sc_prop(ei_hbm, tab_hbm, zero_hbm, out_hbm,
             sidx, didx, rows, acc, stab, gs, ss):
    c = lax.axis_index("c")
    s = lax.axis_index("s")
    w = c * _NS + s

    pltpu.async_copy(ei_hbm.at[0, pl.ds(w * _ET, _ET)], sidx, gs[0])
    pltpu.async_copy(ei_hbm.at[1, pl.ds(w * _ET, _ET)], didx, gs[1])
    pltpu.make_async_copy(ei_hbm.at[0, pl.ds(w * _ET, _ET)], sidx,
                          gs[0]).wait()
    pltpu.make_async_copy(ei_hbm.at[1, pl.ds(w * _ET, _ET)], didx,
                          gs[1]).wait()

    def gather(i, b):
        pltpu.async_copy(stab.at[sidx.at[pl.ds(i * _K, _K)]],
                         rows[b], gs[b])

    def scat(i, b):
        pltpu.async_copy(rows[b], acc.at[didx.at[pl.ds(i * _K, _K)]],
                         ss[b], add=True)

    def wait_gather(b):
        pltpu.make_async_copy(stab.at[sidx.at[pl.ds(0, _K)]],
                              rows[b], gs[b]).wait()

    def wait_scat(b):
        pltpu.make_async_copy(rows[b], acc.at[didx.at[pl.ds(0, _K)]],
                              ss[b]).wait()

    def tail_gather():
        pltpu.async_copy(stab.at[sidx.at[pl.ds(_CF * _K, _TAIL)]],
                         rows[2].at[pl.ds(0, _TAIL)], gs[2])

    def wait_tail_gather():
        pltpu.make_async_copy(stab.at[sidx.at[pl.ds(_CF * _K, _TAIL)]],
                              rows[2].at[pl.ds(0, _TAIL)], gs[2]).wait()

    def tail_scat():
        pltpu.async_copy(rows[2].at[pl.ds(0, _TAIL)],
                         acc.at[didx.at[pl.ds(_CF * _K, _TAIL)]],
                         ss[2], add=True)

    def wait_tail_scat():
        pltpu.make_async_copy(rows[2].at[pl.ds(0, _TAIL)],
                              acc.at[didx.at[pl.ds(_CF * _K, _TAIL)]],
                              ss[2]).wait()

    def step(j, carry):
        for t in range(_NB):
            i = _NB * j + t
            b = t
            bp = (t + _NB - 1) % _NB
            wait_gather(b)
            scat(i, b)
            if t == 0:
                pl.when(j > 0)(lambda bp=bp: wait_scat(bp))
                gather(i + _NB - 1, bp)
            elif t < _NB - 1:
                wait_scat(bp)
                gather(i + _NB - 1, bp)
            else:
                def pre(i=i, bp=bp):
                    wait_scat(bp)
                    gather(i + _NB - 1, bp)
                pl.when(j < _CBF - 1)(pre)
        return carry

    for h in range(2):
        col = pl.ds(h * _DHH, _DHH)
        pltpu.async_copy(zero_hbm, acc.at[pl.ds(s * _RPT, _RPT)], gs[0])
        pltpu.async_copy(tab_hbm.at[pl.ds(s * _TPT, _TPT), col],
                         stab.at[pl.ds(s * _TPT, _TPT)], gs[1])
        pltpu.make_async_copy(zero_hbm, acc.at[pl.ds(s * _RPT, _RPT)],
                              gs[0]).wait()
        pltpu.make_async_copy(tab_hbm.at[pl.ds(s * _TPT, _TPT), col],
                              stab.at[pl.ds(s * _TPT, _TPT)], gs[1]).wait()
        plsc.subcore_barrier()
        for i in range(_NB - 1):
            gather(i, i)
        lax.fori_loop(0, _CBF, step, 0)
        wait_gather(0)
        scat(_CF - 2, 0)
        wait_gather(1)
        scat(_CF - 1, 1)
        wait_scat(2)
        tail_gather()
        wait_tail_gather()
        tail_scat()
        wait_scat(3)
        wait_scat(0)
        wait_scat(1)
        wait_tail_scat()
        plsc.subcore_barrier()
        for k in range(4):
            off = s * _RPT + k * _K
            pltpu.sync_copy(acc.at[pl.ds(off, _K)], rows[k])
            pltpu.sync_copy(rows[k], out_hbm.at[c, pl.ds(off, _K), col])
        off = s * _RPT + 4 * _K
        rem = _RPT - 4 * _K
        pltpu.sync_copy(acc.at[pl.ds(off, rem)], rows[0].at[pl.ds(0, rem)])
        pltpu.sync_copy(rows[0].at[pl.ds(0, rem)],
                        out_hbm.at[c, pl.ds(off, rem), col])
        if h == 0:
            plsc.subcore_barrier()


@functools.partial(
    pl.kernel,
    out_type=jax.ShapeDtypeStruct((_NC, _NPAD, _DW), jnp.float32),
    mesh=_sc_mesh,
    scratch_types=[
        pltpu.VMEM((_ET,), jnp.int32),
        pltpu.VMEM((_K, _DW), jnp.float32),
        pltpu.VMEM_SHARED((_NPAD, _DW), jnp.float32),
        pltpu.SemaphoreType.DMA,
        pltpu.SemaphoreType.DMA,
    ],
    compiler_params=pltpu.CompilerParams(use_tc_tiling_on_sc=False),
)
def _sc_deg(ei_hbm, zero_hbm, ones_hbm, out_hbm, didx, ones_v, acc, s0, s1):
    c = lax.axis_index("c")
    s = lax.axis_index("s")
    w = c * _NS + s

    pltpu.sync_copy(ei_hbm.at[1, pl.ds(w * _ET, _ET)], didx)
    pltpu.sync_copy(zero_hbm, acc.at[pl.ds(s * _RPT, _RPT)])
    pltpu.sync_copy(ones_hbm, ones_v)
    plsc.subcore_barrier()

    def scat(i, sem):
        pltpu.async_copy(ones_v, acc.at[didx.at[pl.ds(i * _K, _K)]],
                         sem, add=True)

    def wait_scat(sem):
        pltpu.make_async_copy(ones_v, acc.at[didx.at[pl.ds(0, _K)]],
                              sem).wait()

    scat(0, s0)
    scat(1, s1)

    def step(j, carry):
        wait_scat(s0)
        scat(2 * j, s0)
        wait_scat(s1)
        scat(2 * j + 1, s1)
        return carry

    lax.fori_loop(1, _CF // 2, step, 0)
    wait_scat(s0)
    pltpu.async_copy(ones_v.at[pl.ds(0, _TAIL)],
                     acc.at[didx.at[pl.ds(_CF * _K, _TAIL)]], s0, add=True)
    wait_scat(s1)
    pltpu.make_async_copy(ones_v.at[pl.ds(0, _TAIL)],
                          acc.at[didx.at[pl.ds(_CF * _K, _TAIL)]], s0).wait()
    plsc.subcore_barrier()
    pltpu.sync_copy(acc.at[pl.ds(s * _RPT, _RPT)],
                    out_hbm.at[c, pl.ds(s * _RPT, _RPT)])


def _tc(body, out_shape, *args):
    return pl.pallas_call(body, out_shape=out_shape)(*args)


def _mm1_body(x, w1, out_hp):
    out_hp[...] = jnp.dot(x[...], w1[...], preferred_element_type=jnp.float32)


def _scale_body(degp, hp, out_u, out_dinv):
    deg = degp[0][:_N, 0:1] + degp[1][:_N, 0:1] + 1.0
    dinv = lax.rsqrt(deg)
    out_u[...] = dinv * hp[...]
    out_dinv[...] = dinv


def _block_tail(s_pair, u, dinv, b, g, be):
    z = dinv[...] * (s_pair[0][:_N] + s_pair[1][:_N] + u[...]) \
        + b[...][None, :]
    a = jnp.where(z >= 0, z, 0.01 * z)
    m = jnp.mean(a, axis=0, keepdims=True)
    v = jnp.mean((a - m) ** 2, axis=0, keepdims=True)
    return g[...][None, :] * (a - m) * lax.rsqrt(v + 1e-5) + be[...][None, :]


def _mid_body(s_pair, u, dinv, b, g, be, wn, out_u):
    h = _block_tail(s_pair, u, dinv, b, g, be)
    hpn = jnp.dot(h, wn[...], preferred_element_type=jnp.float32)
    out_u[...] = dinv[...] * hpn


def _last_block_body(s_pair, u, dinv, b, g, be, out_q):
    h = _block_tail(s_pair, u, dinv, b, g, be)
    out_q[...] = dinv[...] * h


def _heads_body(t_pair, q, dinv, wmu, bmu, wls, bls, out_mu, out_ls):
    r = dinv[...] * (t_pair[0][:_N] + t_pair[1][:_N] + q[...])
    out_mu[...] = jnp.dot(r, wmu[...], preferred_element_type=jnp.float32) \
        + bmu[...][None, :]
    out_ls[...] = jnp.dot(r, wls[...], preferred_element_type=jnp.float32) \
        + bls[...][None, :]


_f32 = jnp.float32


def kernel(x, edge_index, W1, b1, g1, be1, W2, b2, g2, be2,
           W3, b3, g3, be3, Wmu, bmu, Wls, bls):
    zeros32 = jnp.zeros((_RPT, _DHH), _f32)
    zeros16 = jnp.zeros((_RPT, _DW), _f32)
    ones16 = jnp.ones((_K, _DW), _f32)

    nd = jax.ShapeDtypeStruct((_N, _DH), _f32)
    n1 = jax.ShapeDtypeStruct((_N, 1), _f32)

    degp = _sc_deg(edge_index, zeros16, ones16)
    h1p = _tc(_mm1_body, nd, x, W1)
    u1, dinv = _tc(_scale_body, (nd, n1), degp, h1p)
    s1 = _sc_prop(edge_index, u1, zeros32)
    u2 = _tc(_mid_body, nd, s1, u1, dinv, b1, g1, be1, W2)
    s2 = _sc_prop(edge_index, u2, zeros32)
    u3 = _tc(_mid_body, nd, s2, u2, dinv, b2, g2, be2, W3)
    s3 = _sc_prop(edge_index, u3, zeros32)
    q = _tc(_last_block_body, nd, s3, u3, dinv, b3, g3, be3)
    t = _sc_prop(edge_index, q, zeros32)
    no = jax.ShapeDtypeStruct((_N, 128), _f32)
    mu, ls = _tc(_heads_body, (no, no), t, q, dinv, Wmu, bmu, Wls, bls)
    return (mu, ls)
